# baseline (speedup 1.0000x reference)
import functools

import jax
import jax.numpy as jnp
from jax import lax
from jax.experimental import pallas as pl
from jax.experimental.pallas import tpu as pltpu
from jax.experimental.pallas import tpu_sc as plsc

_LANES = 16


def kernel(x, pe_table):
    S, B, D = x.shape
    info = plsc.get_sparse_core_info()
    NC, NS = info.num_cores, info.num_subcores
    NW = NC * NS
    RPW = S // NW
    CH = 8
    NCH = RPW // CH
    mesh = plsc.VectorSubcoreMesh(core_axis_name="c", subcore_axis_name="s")

    @functools.partial(
        pl.kernel,
        mesh=mesh,
        compiler_params=pltpu.CompilerParams(use_tc_tiling_on_sc=True),
        out_type=jax.ShapeDtypeStruct((S, B, D), jnp.float32),
        scratch_types=[
            pltpu.VMEM((3, CH, B, D), jnp.float32),
            pltpu.VMEM((3, CH, D), jnp.float32),
            pltpu.SemaphoreType.DMA,
            pltpu.SemaphoreType.DMA,
            pltpu.SemaphoreType.DMA,
            pltpu.SemaphoreType.DMA,
            pltpu.SemaphoreType.DMA,
            pltpu.SemaphoreType.DMA,
        ],
    )
    def sc_add(x_hbm, pe_hbm, out_hbm, x_buf, pe_buf,
               si0, si1, si2, so0, so1, so2):
        wid = lax.axis_index("s") * NC + lax.axis_index("c")
        base = wid * RPW
        s_in = (si0, si1, si2)
        s_out = (so0, so1, so2)
        NB = 3
        in_h = [None] * NB
        out_h = [None] * NB

        def start_in(c):
            b = c % NB
            rs = base + c * CH
            hx = pltpu.async_copy(
                x_hbm.at[pl.ds(rs, CH)], x_buf.at[b], s_in[b])
            hp = pltpu.async_copy(
                pe_hbm.at[pl.ds(rs, CH)], pe_buf.at[b], s_in[b])
            in_h[b] = (hx, hp)

        start_in(0)
        start_in(1)
        for c in range(NCH):
            b = c % NB
            nxt = c + 2
            if nxt < NCH:
                nb = nxt % NB
                if out_h[nb] is not None:
                    out_h[nb].wait()
                    out_h[nb] = None
                start_in(nxt)
            for h in in_h[b]:
                h.wait()

            def jbody(j, carry, b=b):
                for r in range(CH):
                    pe_v = pe_buf[b, r, pl.ds(j * _LANES, _LANES)]
                    for bb in range(B):
                        sl = x_buf.at[b, r, bb, pl.ds(j * _LANES, _LANES)]
                        sl[...] = sl[...] + pe_v
                return carry

            lax.fori_loop(0, D // _LANES, jbody, 0)
            rs = base + c * CH
            out_h[b] = pltpu.async_copy(
                x_buf.at[b], out_hbm.at[pl.ds(rs, CH)], s_out[b])
        for h in out_h:
            if h is not None:
                h.wait()

    return sc_add(x, pe_table[:S])


# speedup vs baseline: 1.2190x; 1.2190x over previous
import functools

import jax
import jax.numpy as jnp
from jax import lax
from jax.experimental import pallas as pl
from jax.experimental.pallas import tpu as pltpu
from jax.experimental.pallas import tpu_sc as plsc

_LANES = 16


def kernel(x, pe_table):
    S, B, D = x.shape
    info = plsc.get_sparse_core_info()
    NC, NS = info.num_cores, info.num_subcores
    NW = NC * NS
    RPW = S // NW
    CH = 8
    NCH = RPW // CH
    mesh = plsc.VectorSubcoreMesh(core_axis_name="c", subcore_axis_name="s")

    @functools.partial(
        pl.kernel,
        mesh=mesh,
        compiler_params=pltpu.CompilerParams(use_tc_tiling_on_sc=True),
        out_type=jax.ShapeDtypeStruct((S, B, D), jnp.float32),
        scratch_types=[
            pltpu.VMEM((3, CH, B, D), jnp.float32),
            pltpu.VMEM((3, CH, D), jnp.float32),
            pltpu.SemaphoreType.DMA,
            pltpu.SemaphoreType.DMA,
            pltpu.SemaphoreType.DMA,
            pltpu.SemaphoreType.DMA,
            pltpu.SemaphoreType.DMA,
            pltpu.SemaphoreType.DMA,
        ],
    )
    def sc_add(x_hbm, pe_hbm, out_hbm, x_buf, pe_buf,
               si0, si1, si2, so0, so1, so2):
        wid = lax.axis_index("s") * NC + lax.axis_index("c")
        base = wid * RPW
        s_in = (si0, si1, si2)
        s_out = (so0, so1, so2)
        NB = 3
        in_h = [None] * NB
        out_h = [None] * NB

        def start_in(c):
            b = c % NB
            rs = base + c * CH
            hx = pltpu.async_copy(
                x_hbm.at[pl.ds(rs, CH)], x_buf.at[b], s_in[b])
            hp = pltpu.async_copy(
                pe_hbm.at[pl.ds(rs, CH)], pe_buf.at[b], s_in[b])
            in_h[b] = (hx, hp)

        start_in(0)
        start_in(1)
        for c in range(NCH):
            b = c % NB
            nxt = c + 2
            if nxt < NCH:
                nb = nxt % NB
                if out_h[nb] is not None:
                    out_h[nb].wait()
                    out_h[nb] = None
                start_in(nxt)
            for h in in_h[b]:
                h.wait()

            @plsc.parallel_loop(0, D // _LANES, unroll=2)
            def jbody(j, b=b):
                for r in range(CH):
                    pe_v = pe_buf[b, r, pl.ds(j * _LANES, _LANES)]
                    for bb in range(B):
                        plsc.addupdate(
                            x_buf.at[b, r, bb, pl.ds(j * _LANES, _LANES)],
                            pe_v)
            rs = base + c * CH
            out_h[b] = pltpu.async_copy(
                x_buf.at[b], out_hbm.at[pl.ds(rs, CH)], s_out[b])
        for h in out_h:
            if h is not None:
                h.wait()

    return sc_add(x, pe_table[:S])


# speedup vs baseline: 1.2270x; 1.0066x over previous
import functools

import jax
import jax.numpy as jnp
from jax import lax
from jax.experimental import pallas as pl
from jax.experimental.pallas import tpu as pltpu
from jax.experimental.pallas import tpu_sc as plsc

_LANES = 16


def kernel(x, pe_table):
    S, B, D = x.shape
    info = plsc.get_sparse_core_info()
    NC, NS = info.num_cores, info.num_subcores
    NW = NC * NS
    RPW = S // NW
    CH = 8
    NCH = RPW // CH
    mesh = plsc.VectorSubcoreMesh(core_axis_name="c", subcore_axis_name="s")

    @functools.partial(
        pl.kernel,
        mesh=mesh,
        compiler_params=pltpu.CompilerParams(use_tc_tiling_on_sc=True),
        out_type=jax.ShapeDtypeStruct((S, B, D), jnp.float32),
        scratch_types=[
            pltpu.VMEM((3, CH, B, D), jnp.float32),
            pltpu.VMEM((3, CH, D), jnp.float32),
            pltpu.SemaphoreType.DMA,
            pltpu.SemaphoreType.DMA,
            pltpu.SemaphoreType.DMA,
            pltpu.SemaphoreType.DMA,
            pltpu.SemaphoreType.DMA,
            pltpu.SemaphoreType.DMA,
        ],
    )
    def sc_add(x_hbm, pe_hbm, out_hbm, x_buf, pe_buf,
               si0, si1, si2, so0, so1, so2):
        wid = lax.axis_index("s") * NC + lax.axis_index("c")
        base = wid * RPW
        s_in = (si0, si1, si2)
        s_out = (so0, so1, so2)
        NB = 3
        in_h = [None] * NB
        out_h = [None] * NB

        def start_in(c):
            b = c % NB
            rs = base + c * CH
            hx = pltpu.async_copy(
                x_hbm.at[pl.ds(rs, CH)], x_buf.at[b], s_in[b])
            hp = pltpu.async_copy(
                pe_hbm.at[pl.ds(rs, CH)], pe_buf.at[b], s_in[b])
            in_h[b] = (hx, hp)

        start_in(0)
        start_in(1)
        for c in range(NCH):
            b = c % NB
            nxt = c + 2
            if nxt < NCH:
                nb = nxt % NB
                if out_h[nb] is not None:
                    out_h[nb].wait()
                    out_h[nb] = None
                start_in(nxt)
            for h in in_h[b]:
                h.wait()

            @plsc.parallel_loop(0, D // _LANES, unroll=4)
            def jbody(j, b=b):
                for r in range(CH):
                    pe_v = pe_buf[b, r, pl.ds(j * _LANES, _LANES)]
                    for bb in range(B):
                        plsc.addupdate(
                            x_buf.at[b, r, bb, pl.ds(j * _LANES, _LANES)],
                            pe_v)
            rs = base + c * CH
            out_h[b] = pltpu.async_copy(
                x_buf.at[b], out_hbm.at[pl.ds(rs, CH)], s_out[b])
        for h in out_h:
            if h is not None:
                h.wait()

    return sc_add(x, pe_table[:S])


# speedup vs baseline: 1.2400x; 1.0106x over previous
import functools

import jax
import jax.numpy as jnp
from jax import lax
from jax.experimental import pallas as pl
from jax.experimental.pallas import tpu as pltpu
from jax.experimental.pallas import tpu_sc as plsc

_LANES = 16


def kernel(x, pe_table):
    S, B, D = x.shape
    info = plsc.get_sparse_core_info()
    NC, NS = info.num_cores, info.num_subcores
    NW = NC * NS
    RPW = S // NW
    CH = 4
    NCH = RPW // CH
    mesh = plsc.VectorSubcoreMesh(core_axis_name="c", subcore_axis_name="s")

    @functools.partial(
        pl.kernel,
        mesh=mesh,
        compiler_params=pltpu.CompilerParams(use_tc_tiling_on_sc=True),
        out_type=jax.ShapeDtypeStruct((S, B, D), jnp.float32),
        scratch_types=[
            pltpu.VMEM((6, CH, B, D), jnp.float32),
            pltpu.VMEM((6, CH, D), jnp.float32),
            pltpu.SemaphoreType.DMA,
            pltpu.SemaphoreType.DMA,
            pltpu.SemaphoreType.DMA,
            pltpu.SemaphoreType.DMA,
            pltpu.SemaphoreType.DMA,
            pltpu.SemaphoreType.DMA,
            pltpu.SemaphoreType.DMA,
            pltpu.SemaphoreType.DMA,
            pltpu.SemaphoreType.DMA,
            pltpu.SemaphoreType.DMA,
            pltpu.SemaphoreType.DMA,
            pltpu.SemaphoreType.DMA,
        ],
    )
    def sc_add(x_hbm, pe_hbm, out_hbm, x_buf, pe_buf,
               si0, si1, si2, si3, si4, si5, so0, so1, so2, so3, so4, so5):
        wid = lax.axis_index("s") * NC + lax.axis_index("c")
        base = wid * RPW
        s_in = (si0, si1, si2, si3, si4, si5)
        s_out = (so0, so1, so2, so3, so4, so5)
        NB = 6
        in_h = [None] * NB
        out_h = [None] * NB

        def start_in(c):
            b = c % NB
            rs = base + c * CH
            hx = pltpu.async_copy(
                x_hbm.at[pl.ds(rs, CH)], x_buf.at[b], s_in[b])
            hp = pltpu.async_copy(
                pe_hbm.at[pl.ds(rs, CH)], pe_buf.at[b], s_in[b])
            in_h[b] = (hx, hp)

        for p in range(min(4, NCH)):
            start_in(p)
        for c in range(NCH):
            b = c % NB
            nxt = c + 4
            if nxt < NCH:
                nb = nxt % NB
                if out_h[nb] is not None:
                    out_h[nb].wait()
                    out_h[nb] = None
                start_in(nxt)
            for h in in_h[b]:
                h.wait()

            @plsc.parallel_loop(0, D // _LANES, unroll=4)
            def jbody(j, b=b):
                for r in range(CH):
                    pe_v = pe_buf[b, r, pl.ds(j * _LANES, _LANES)]
                    for bb in range(B):
                        plsc.addupdate(
                            x_buf.at[b, r, bb, pl.ds(j * _LANES, _LANES)],
                            pe_v)
            rs = base + c * CH
            out_h[b] = pltpu.async_copy(
                x_buf.at[b], out_hbm.at[pl.ds(rs, CH)], s_out[b])
        for h in out_h:
            if h is not None:
                h.wait()

    return sc_add(x, pe_table[:S])


# speedup vs baseline: 1.2491x; 1.0073x over previous
import functools

import jax
import jax.numpy as jnp
from jax import lax
from jax.experimental import pallas as pl
from jax.experimental.pallas import tpu as pltpu
from jax.experimental.pallas import tpu_sc as plsc

_LANES = 16


def kernel(x, pe_table):
    S, B, D = x.shape
    info = plsc.get_sparse_core_info()
    NC, NS = info.num_cores, info.num_subcores
    NW = NC * NS
    RPW = S // NW
    CH = 4
    NCH = RPW // CH
    mesh = plsc.VectorSubcoreMesh(core_axis_name="c", subcore_axis_name="s")

    @functools.partial(
        pl.kernel,
        mesh=mesh,
        compiler_params=pltpu.CompilerParams(use_tc_tiling_on_sc=True),
        out_type=jax.ShapeDtypeStruct((S, B, D), jnp.float32),
        scratch_types=[
            pltpu.VMEM((6, CH, B, D), jnp.float32),
            pltpu.VMEM((6, CH, D), jnp.float32),
            pltpu.SemaphoreType.DMA,
            pltpu.SemaphoreType.DMA,
            pltpu.SemaphoreType.DMA,
            pltpu.SemaphoreType.DMA,
            pltpu.SemaphoreType.DMA,
            pltpu.SemaphoreType.DMA,
            pltpu.SemaphoreType.DMA,
            pltpu.SemaphoreType.DMA,
            pltpu.SemaphoreType.DMA,
            pltpu.SemaphoreType.DMA,
            pltpu.SemaphoreType.DMA,
            pltpu.SemaphoreType.DMA,
        ],
    )
    def sc_add(x_hbm, pe_hbm, out_hbm, x_buf, pe_buf,
               si0, si1, si2, si3, si4, si5, so0, so1, so2, so3, so4, so5):
        wid = lax.axis_index("s") * NC + lax.axis_index("c")
        base = wid * RPW
        s_in = (si0, si1, si2, si3, si4, si5)
        s_out = (so0, so1, so2, so3, so4, so5)
        NB = 6
        in_h = [None] * NB
        out_h = [None] * NB

        def start_in(c):
            b = c % NB
            rs = base + c * CH
            hx = pltpu.async_copy(
                x_hbm.at[pl.ds(rs, CH)], x_buf.at[b], s_in[b])
            hp = pltpu.async_copy(
                pe_hbm.at[pl.ds(rs, CH)], pe_buf.at[b], s_in[b])
            in_h[b] = (hx, hp)

        for p in range(min(4, NCH)):
            start_in(p)
        for c in range(NCH):
            b = c % NB
            nxt = c + 4
            if nxt < NCH:
                nb = nxt % NB
                if out_h[nb] is not None:
                    for h in out_h[nb]:
                        h.wait()
                    out_h[nb] = None
                start_in(nxt)
            for h in in_h[b]:
                h.wait()

            rs = base + c * CH
            hs = []
            for half in range(2):
                HR = CH // 2

                @plsc.parallel_loop(0, D // _LANES, unroll=4)
                def jbody(j, b=b, half=half):
                    for r in range(half * HR, (half + 1) * HR):
                        pe_v = pe_buf[b, r, pl.ds(j * _LANES, _LANES)]
                        for bb in range(B):
                            plsc.addupdate(
                                x_buf.at[b, r, bb, pl.ds(j * _LANES, _LANES)],
                                pe_v)
                hs.append(pltpu.async_copy(
                    x_buf.at[b, pl.ds(half * HR, HR)],
                    out_hbm.at[pl.ds(rs + half * HR, HR)], s_out[b]))
            out_h[b] = hs
        for hb in out_h:
            if hb is not None:
                for h in hb:
                    h.wait()

    return sc_add(x, pe_table[:S])


# speedup vs baseline: 1.2618x; 1.0101x over previous
import functools

import jax
import jax.numpy as jnp
from jax import lax
from jax.experimental import pallas as pl
from jax.experimental.pallas import tpu as pltpu
from jax.experimental.pallas import tpu_sc as plsc

_LANES = 16


def kernel(x, pe_table):
    S, B, D = x.shape
    info = plsc.get_sparse_core_info()
    NC, NS = info.num_cores, info.num_subcores
    NW = NC * NS
    RPW = S // NW
    CH = 4
    NCH = RPW // CH
    mesh = plsc.VectorSubcoreMesh(core_axis_name="c", subcore_axis_name="s")

    @functools.partial(
        pl.kernel,
        mesh=mesh,
        compiler_params=pltpu.CompilerParams(use_tc_tiling_on_sc=True),
        out_type=jax.ShapeDtypeStruct((S, B, D), jnp.float32),
        scratch_types=[
            pltpu.VMEM((6, CH, B, D), jnp.float32),
            pltpu.VMEM((6, CH, D), jnp.float32),
            pltpu.SemaphoreType.DMA,
            pltpu.SemaphoreType.DMA,
            pltpu.SemaphoreType.DMA,
            pltpu.SemaphoreType.DMA,
            pltpu.SemaphoreType.DMA,
            pltpu.SemaphoreType.DMA,
            pltpu.SemaphoreType.DMA,
            pltpu.SemaphoreType.DMA,
            pltpu.SemaphoreType.DMA,
            pltpu.SemaphoreType.DMA,
            pltpu.SemaphoreType.DMA,
            pltpu.SemaphoreType.DMA,
        ],
    )
    def sc_add(x_hbm, pe_hbm, out_hbm, x_buf, pe_buf,
               si0, si1, si2, si3, si4, si5, so0, so1, so2, so3, so4, so5):
        wid = lax.axis_index("s") * NC + lax.axis_index("c")
        base = wid * RPW
        s_in = (si0, si1, si2, si3, si4, si5)
        s_out = (so0, so1, so2, so3, so4, so5)
        NB = 6
        in_h = [None] * NB
        out_h = [None] * NB

        def start_in(c):
            b = c % NB
            rs = base + c * CH
            hx = pltpu.async_copy(
                x_hbm.at[pl.ds(rs, CH)], x_buf.at[b], s_in[b])
            hp = pltpu.async_copy(
                pe_hbm.at[pl.ds(rs, CH)], pe_buf.at[b], s_in[b])
            in_h[b] = (hx, hp)

        for p in range(min(5, NCH)):
            start_in(p)
        for c in range(NCH):
            b = c % NB
            nxt = c + 5
            if nxt < NCH:
                nb = nxt % NB
                if out_h[nb] is not None:
                    for h in out_h[nb]:
                        h.wait()
                    out_h[nb] = None
                start_in(nxt)
            for h in in_h[b]:
                h.wait()

            rs = base + c * CH
            hs = []
            for half in range(2):
                HR = CH // 2

                @plsc.parallel_loop(0, D // _LANES, unroll=4)
                def jbody(j, b=b, half=half):
                    for r in range(half * HR, (half + 1) * HR):
                        pe_v = pe_buf[b, r, pl.ds(j * _LANES, _LANES)]
                        for bb in range(B):
                            plsc.addupdate(
                                x_buf.at[b, r, bb, pl.ds(j * _LANES, _LANES)],
                                pe_v)
                hs.append(pltpu.async_copy(
                    x_buf.at[b, pl.ds(half * HR, HR)],
                    out_hbm.at[pl.ds(rs + half * HR, HR)], s_out[b]))
            out_h[b] = hs
        for hb in out_h:
            if hb is not None:
                for h in hb:
                    h.wait()

    return sc_add(x, pe_table[:S])
